# baseline (speedup 1.0000x reference)
import jax
import jax.numpy as jnp
from jax.experimental import pallas as pl
from jax.experimental.pallas import tpu as pltpu

MAX_TILES = 4


def _pe_kernel(th_ref, tw_ref, coef_ref, a_ref, x_ref, lpe_ref, gpe_ref, o_ref,
               s_ref, cur_ref):
    b = pl.program_id(0)
    t = pl.program_id(1)
    a = a_ref[0]
    c = coef_ref[b, t]

    @pl.when((b == 0) & (t == 0))
    def _():
        s_ref[...] = a * lpe_ref[...]
        cur_ref[0] = -1

    need = jnp.where(c == 0.0, -1, th_ref[b, t] * MAX_TILES + tw_ref[b, t])

    @pl.when(need != cur_ref[0])
    def _():
        s_ref[...] = a * lpe_ref[...]

        @pl.when(need >= 0)
        def _():
            s_ref[...] += c * gpe_ref[0, 0, :, :]

        cur_ref[0] = need

    o_ref[0, 0, :, :] = x_ref[0, 0, :, :] + s_ref[:, :]


def kernel(x, aspect_ratio, local_pe, global_pe, gate):
    B, T, N, D = x.shape

    g = jnp.tanh(gate[0].astype(jnp.float32))
    a = (1.0 - g).reshape(1)

    h = aspect_ratio[:, 0].astype(jnp.int32)
    w = aspect_ratio[:, 1].astype(jnp.int32)
    w_safe = jnp.maximum(w, 1)
    t = jnp.arange(T, dtype=jnp.int32)
    th = jnp.clip(t[None, :] // w_safe[:, None], 0, MAX_TILES - 1)
    tw = jnp.clip(t[None, :] % w_safe[:, None], 0, MAX_TILES - 1)
    mask = t[None, :] < (h * w)[:, None]
    coef = jnp.where(mask, g, 0.0).astype(jnp.float32)
    th = jnp.where(mask, th, 0).astype(jnp.int32)
    tw = jnp.where(mask, tw, 0).astype(jnp.int32)

    grid_spec = pltpu.PrefetchScalarGridSpec(
        num_scalar_prefetch=4,
        grid=(B, T),
        in_specs=[
            pl.BlockSpec((1, 1, N, D), lambda b, t, th, tw, cf, av: (b, t, 0, 0)),
            pl.BlockSpec((N, D), lambda b, t, th, tw, cf, av: (0, 0)),
            pl.BlockSpec(
                (1, 1, N, D),
                lambda b, t, th, tw, cf, av: (th[b, t], tw[b, t], 0, 0),
            ),
        ],
        out_specs=pl.BlockSpec((1, 1, N, D), lambda b, t, th, tw, cf, av: (b, t, 0, 0)),
        scratch_shapes=[
            pltpu.VMEM((N, D), jnp.float32),
            pltpu.SMEM((1,), jnp.int32),
        ],
    )

    return pl.pallas_call(
        _pe_kernel,
        grid_spec=grid_spec,
        out_shape=jax.ShapeDtypeStruct(x.shape, x.dtype),
    )(th, tw, coef, a, x, local_pe, global_pe)


# speedup vs baseline: 1.1244x; 1.1244x over previous
import jax
import jax.numpy as jnp
from jax import lax
from jax.experimental import pallas as pl
from jax.experimental.pallas import tpu as pltpu

MAX_TILES = 4


def _fast_kernel(a_ref, x_ref, lpe_ref, o_ref):
    a = a_ref[0]
    o_ref[0, 0, :, :] = x_ref[0, 0, :, :] + a * lpe_ref[:, :]


def _fast_call(a, x, local_pe):
    B, T, N, D = x.shape
    grid_spec = pltpu.PrefetchScalarGridSpec(
        num_scalar_prefetch=1,
        grid=(B, T),
        in_specs=[
            pl.BlockSpec((1, 1, N, D), lambda b, t, av: (b, t, 0, 0)),
            pl.BlockSpec((N, D), lambda b, t, av: (0, 0)),
        ],
        out_specs=pl.BlockSpec((1, 1, N, D), lambda b, t, av: (b, t, 0, 0)),
    )
    return pl.pallas_call(
        _fast_kernel,
        grid_spec=grid_spec,
        out_shape=jax.ShapeDtypeStruct(x.shape, x.dtype),
    )(a, x, local_pe)


def _general_kernel(th_ref, tw_ref, coef_ref, a_ref, x_ref, lpe_ref, gpe_ref,
                    o_ref, gbuf_ref, cur_ref, sem):
    b = pl.program_id(0)
    t = pl.program_id(1)
    a = a_ref[0]
    c = coef_ref[b, t]

    @pl.when((b == 0) & (t == 0))
    def _():
        cur_ref[0] = -1

    @pl.when(c == 0.0)
    def _():
        o_ref[0, 0, :, :] = x_ref[0, 0, :, :] + a * lpe_ref[:, :]

    @pl.when(c != 0.0)
    def _():
        i = th_ref[b, t]
        j = tw_ref[b, t]
        key = i * MAX_TILES + j

        @pl.when(cur_ref[0] != key)
        def _():
            pltpu.make_async_copy(gpe_ref.at[i, j], gbuf_ref, sem).start()
            pltpu.make_async_copy(gpe_ref.at[i, j], gbuf_ref, sem).wait()
            cur_ref[0] = key

        o_ref[0, 0, :, :] = (
            x_ref[0, 0, :, :] + a * lpe_ref[:, :] + c * gbuf_ref[:, :]
        )


def _general_call(th, tw, coef, a, x, local_pe, global_pe):
    B, T, N, D = x.shape
    grid_spec = pltpu.PrefetchScalarGridSpec(
        num_scalar_prefetch=4,
        grid=(B, T),
        in_specs=[
            pl.BlockSpec((1, 1, N, D), lambda b, t, th, tw, cf, av: (b, t, 0, 0)),
            pl.BlockSpec((N, D), lambda b, t, th, tw, cf, av: (0, 0)),
            pl.BlockSpec(memory_space=pl.ANY),
        ],
        out_specs=pl.BlockSpec((1, 1, N, D), lambda b, t, th, tw, cf, av: (b, t, 0, 0)),
        scratch_shapes=[
            pltpu.VMEM((N, D), jnp.float32),
            pltpu.SMEM((1,), jnp.int32),
            pltpu.SemaphoreType.DMA,
        ],
    )
    return pl.pallas_call(
        _general_kernel,
        grid_spec=grid_spec,
        out_shape=jax.ShapeDtypeStruct(x.shape, x.dtype),
    )(th, tw, coef, a, x, local_pe, global_pe)


def kernel(x, aspect_ratio, local_pe, global_pe, gate):
    B, T, N, D = x.shape

    g = jnp.tanh(gate[0].astype(jnp.float32))
    a = (1.0 - g).reshape(1)

    h = aspect_ratio[:, 0].astype(jnp.int32)
    w = aspect_ratio[:, 1].astype(jnp.int32)
    w_safe = jnp.maximum(w, 1)
    t = jnp.arange(T, dtype=jnp.int32)
    th = jnp.clip(t[None, :] // w_safe[:, None], 0, MAX_TILES - 1)
    tw = jnp.clip(t[None, :] % w_safe[:, None], 0, MAX_TILES - 1)
    mask = t[None, :] < (h * w)[:, None]
    coef = jnp.where(mask, g, 0.0).astype(jnp.float32)
    th = jnp.where(mask, th, 0).astype(jnp.int32)
    tw = jnp.where(mask, tw, 0).astype(jnp.int32)

    no_global = jnp.all(coef == 0.0)
    return lax.cond(
        no_global,
        lambda ops: _fast_call(ops[3], ops[4], ops[5]),
        lambda ops: _general_call(*ops),
        (th, tw, coef, a, x, local_pe, global_pe),
    )
